# Initial kernel scaffold; baseline (speedup 1.0000x reference)
import jax
import jax.numpy as jnp
from jax.experimental import pallas as pl


def kernel(src_indices, tgt_indices, src_table, tgt_table):
    raise NotImplementedError("write your pallas kernel here")


# speedup vs baseline: 4.7865x; 4.7865x over previous
import jax
import jax.numpy as jnp
from jax import lax
from jax.experimental import pallas as pl
from jax.experimental.pallas import tpu as pltpu, tpu_sc as plsc

DIM = 64
NC, NS = 2, 16
NW = NC * NS
IDX_SEG = 128
K = 5
CH = K * IDX_SEG
N = 4096 * 50
NCHUNK = N // (NW * CH)


def _body(src_tbl, tgt_tbl, src_idx, tgt_idx, src_out, tgt_out,
          idx_v, rows_v, sem):
    wid = lax.axis_index("s") * NC + lax.axis_index("c")
    for idx_h, tbl_h, out_h in ((src_idx, src_tbl, src_out),
                                (tgt_idx, tgt_tbl, tgt_out)):
        def chunk_body(c, carry, idx_h=idx_h, tbl_h=tbl_h, out_h=out_h):
            chunk = wid * NCHUNK + c
            pltpu.sync_copy(idx_h.at[chunk], idx_v)
            cps = [
                pltpu.async_copy(tbl_h.at[idx_v.at[j]],
                                 rows_v.at[pl.ds(j * IDX_SEG, IDX_SEG)],
                                 sem)
                for j in range(K)
            ]
            for cp in cps:
                cp.wait()
            pltpu.sync_copy(rows_v, out_h.at[pl.ds(chunk * CH, CH)])
            return carry
        lax.fori_loop(0, NCHUNK, chunk_body, 0)


@jax.jit
def kernel(src_indices, tgt_indices, src_table, tgt_table):
    b, l = src_indices.shape
    si = src_indices.astype(jnp.int32).reshape(NW * NCHUNK, K, IDX_SEG)
    ti = tgt_indices.astype(jnp.int32).reshape(NW * NCHUNK, K, IDX_SEG)
    f = pl.kernel(
        _body,
        out_type=(
            jax.ShapeDtypeStruct((N, DIM), jnp.float32),
            jax.ShapeDtypeStruct((N, DIM), jnp.float32),
        ),
        mesh=plsc.VectorSubcoreMesh(core_axis_name="c", subcore_axis_name="s"),
        scratch_types=[
            pltpu.VMEM((K, IDX_SEG), jnp.int32),
            pltpu.VMEM((CH, DIM), jnp.float32),
            pltpu.SemaphoreType.DMA,
        ],
        compiler_params=pltpu.CompilerParams(use_tc_tiling_on_sc=False),
    )
    src_out, tgt_out = f(src_table, tgt_table, si, ti)
    return (src_out.reshape(b, l, DIM), tgt_out.reshape(b, l, DIM))


# speedup vs baseline: 5.0326x; 1.0514x over previous
import jax
import jax.numpy as jnp
from jax import lax
from jax.experimental import pallas as pl
from jax.experimental.pallas import tpu as pltpu, tpu_sc as plsc

DIM = 64
NC, NS = 2, 16
NW = NC * NS
IDX_SEG = 128
K = 2
CH = K * IDX_SEG
N = 4096 * 50
NCHUNK = N // (NW * CH)
PADC = 2


def _body(src_tbl, tgt_tbl, src_idx, tgt_idx, src_out, tgt_out,
          idx_s, idx_t, rows_s, rows_t, sg_s, sg_t, so_s, so_t, si_s, si_t):
    wid = lax.axis_index("s") * NC + lax.axis_index("c")
    base = wid * NCHUNK

    streams = (
        (src_idx, src_tbl, src_out, idx_s, rows_s, sg_s, so_s, si_s),
        (tgt_idx, tgt_tbl, tgt_out, idx_t, rows_t, sg_t, so_t, si_t),
    )

    def fire_gathers(tbl_h, idxb, rowsb, b, sem):
        for j in range(K):
            pltpu.async_copy(tbl_h.at[idxb.at[b, j]],
                             rowsb.at[b, pl.ds(j * IDX_SEG, IDX_SEG)],
                             sem)

    for idx_h, tbl_h, out_h, idxb, rowsb, sg, so, si in streams:
        pltpu.sync_copy(idx_h.at[base], idxb.at[0])
        fire_gathers(tbl_h, idxb, rowsb, 0, sg)
        pltpu.async_copy(idx_h.at[base + 1], idxb.at[1], si)

    def loop_body(c, carry):
        b = lax.rem(c, 2)
        nb = 1 - b
        for idx_h, tbl_h, out_h, idxb, rowsb, sg, so, si in streams:
            pltpu.make_async_copy(tbl_h.at[pl.ds(0, CH)], rowsb.at[b], sg).wait()
            @pl.when(c > 0)
            def _():
                pltpu.make_async_copy(rowsb.at[nb],
                                      out_h.at[pl.ds(0, CH)], so).wait()
            pltpu.async_copy(rowsb.at[b],
                             out_h.at[pl.ds((base + c) * CH, CH)], so)
            pltpu.make_async_copy(idx_h.at[0], idxb.at[nb], si).wait()
            pltpu.async_copy(idx_h.at[base + c + 2], idxb.at[b], si)
            @pl.when(c + 1 < NCHUNK)
            def _():
                fire_gathers(tbl_h, idxb, rowsb, nb, sg)
        return carry

    lax.fori_loop(0, NCHUNK, loop_body, 0)

    last_b = (NCHUNK - 1) % 2
    for idx_h, tbl_h, out_h, idxb, rowsb, sg, so, si in streams:
        pltpu.make_async_copy(rowsb.at[last_b], out_h.at[pl.ds(0, CH)], so).wait()
        pltpu.make_async_copy(idx_h.at[0], idxb.at[last_b], si).wait()


@jax.jit
def kernel(src_indices, tgt_indices, src_table, tgt_table):
    b, l = src_indices.shape
    si = src_indices.astype(jnp.int32).reshape(NW * NCHUNK, K, IDX_SEG)
    ti = tgt_indices.astype(jnp.int32).reshape(NW * NCHUNK, K, IDX_SEG)
    pad = jnp.zeros((PADC, K, IDX_SEG), jnp.int32)
    si = jnp.concatenate([si, pad], axis=0)
    ti = jnp.concatenate([ti, pad], axis=0)
    f = pl.kernel(
        _body,
        out_type=(
            jax.ShapeDtypeStruct((N, DIM), jnp.float32),
            jax.ShapeDtypeStruct((N, DIM), jnp.float32),
        ),
        mesh=plsc.VectorSubcoreMesh(core_axis_name="c", subcore_axis_name="s"),
        scratch_types=[
            pltpu.VMEM((2, K, IDX_SEG), jnp.int32),
            pltpu.VMEM((2, K, IDX_SEG), jnp.int32),
            pltpu.VMEM((2, CH, DIM), jnp.float32),
            pltpu.VMEM((2, CH, DIM), jnp.float32),
            pltpu.SemaphoreType.DMA,
            pltpu.SemaphoreType.DMA,
            pltpu.SemaphoreType.DMA,
            pltpu.SemaphoreType.DMA,
            pltpu.SemaphoreType.DMA,
            pltpu.SemaphoreType.DMA,
        ],
        compiler_params=pltpu.CompilerParams(use_tc_tiling_on_sc=False),
    )
    src_out, tgt_out = f(src_table, tgt_table, si, ti)
    return (src_out.reshape(b, l, DIM), tgt_out.reshape(b, l, DIM))
